# Initial kernel scaffold; baseline (speedup 1.0000x reference)
import jax
import jax.numpy as jnp
from jax.experimental import pallas as pl


def kernel(x, edge_attr, params, edge_index_bond, edge_index_complete):
    raise NotImplementedError("write your pallas kernel here")


# speedup vs baseline: 2.2559x; 2.2559x over previous
import functools
import jax
import jax.numpy as jnp
from jax import lax
from jax.experimental import pallas as pl
from jax.experimental.pallas import tpu as pltpu
from jax.experimental.pallas import tpu_sc as plsc

N = 10000
E = 320000
H = 128
AVG_D_LOG = 1.0

NSC = 32
NPART = 96
NPB = 112
NPAD = NPART * NPB
CH = 96
CHB = CH + 16
CH2 = 400
EPW = E // NSC
RPW = 192
PADN = N + 1024
PADE = E + 256

F32MAX = 3.4e38


def _align8(i):
    return (i // 8) * 8


def _extract_i32(vref, idx):
    return vref[pl.ds(idx, 16)][0]


def _sc_stats_body(nsrc, *refs):
    if nsrc == 3:
        (a_hbm, b_hbm, c_hbm, srcs_hbm, dsts_hbm, perm_hbm, rp_hbm,
         out_hbm, deg_hbm,
         ia_v, ib_v, ic_v, bufa_v, bufb_v, bufc_v, rp_v, acc_v, deg_v) = refs
    else:
        (c_hbm, dsts_hbm, perm_hbm, rp_hbm, out_hbm, deg_hbm,
         ib_v, ic_v, bufc_v, rp_v, acc_v, deg_v) = refs

    w = lax.axis_index("s") * 2 + lax.axis_index("c")

    def round_body(r, _0):
        part = w * 3 + r
        n0 = part * NPB
        n1 = jnp.minimum(jnp.maximum(N, n0), n0 + NPB)
        rp_base = _align8(n0)
        pltpu.sync_copy(rp_hbm.at[pl.ds(rp_base, RPW)], rp_v)

        def rp(i, _rp_base=rp_base):
            return _extract_i32(rp_v, i - _rp_base)

        e_lo = rp(n0)
        e_hi = rp(n1)
        e_al = _align8(e_lo)
        nchunks = (e_hi - e_al + CH - 1) // CH

        def init_body(i, _):
            for t in range(8):
                s = pl.ds(16 * t, 16)
                acc_v[i, 0, s] = jnp.zeros((16,), jnp.float32)
                acc_v[i, 1, s] = jnp.zeros((16,), jnp.float32)
                acc_v[i, 2, s] = jnp.full((16,), -F32MAX, jnp.float32)
                acc_v[i, 3, s] = jnp.full((16,), F32MAX, jnp.float32)
            deg_v[i, pl.ds(0, 16)] = jnp.zeros((16,), jnp.float32)
            return 0

        lax.fori_loop(0, NPB, init_body, 0)

        def chunk_body(k, _, _n0=n0, _e_lo=e_lo, _e_hi=e_hi, _e_al=e_al):
            base = _e_al + k * CH
            if nsrc == 3:
                pltpu.sync_copy(srcs_hbm.at[pl.ds(base, CHB)], ia_v)
                pltpu.sync_copy(dsts_hbm.at[pl.ds(base, CHB)], ib_v)
                pltpu.sync_copy(perm_hbm.at[pl.ds(base, CHB)], ic_v)
                pltpu.sync_copy(a_hbm.at[ia_v], bufa_v)
                pltpu.sync_copy(b_hbm.at[ib_v], bufb_v)
                pltpu.sync_copy(c_hbm.at[ic_v], bufc_v)
            else:
                pltpu.sync_copy(dsts_hbm.at[pl.ds(base, CHB)], ib_v)
                pltpu.sync_copy(perm_hbm.at[pl.ds(base, CHB)], ic_v)
                pltpu.sync_copy(c_hbm.at[ic_v], bufc_v)
            j0 = jnp.maximum(_e_lo - base, 0)
            j1 = jnp.minimum(_e_hi - base, CH)

            def edge_body(j, _2):
                off = _extract_i32(ib_v, j) - _n0
                ones = jnp.full((16,), 1.0, jnp.float32)
                deg_v[off, pl.ds(0, 16)] = deg_v[off, pl.ds(0, 16)] + ones
                for t in range(8):
                    s = pl.ds(16 * t, 16)
                    m = bufc_v[j, s]
                    if nsrc == 3:
                        m = m + bufa_v[j, s]
                        m = m + bufb_v[j, s]
                    acc_v[off, 0, s] = acc_v[off, 0, s] + m
                    acc_v[off, 1, s] = acc_v[off, 1, s] + m * m
                    acc_v[off, 2, s] = jnp.maximum(acc_v[off, 2, s], m)
                    acc_v[off, 3, s] = jnp.minimum(acc_v[off, 3, s], m)
                return 0

            lax.fori_loop(j0, j1, edge_body, 0)
            return 0

        lax.fori_loop(0, nchunks, chunk_body, 0)
        pltpu.sync_copy(acc_v, out_hbm.at[pl.ds(n0, NPB)])
        pltpu.sync_copy(deg_v, deg_hbm.at[pl.ds(n0, NPB)])
        return 0

    lax.fori_loop(0, 3, round_body, 0)


def _make_sc_stats(nsrc):
    mesh = plsc.VectorSubcoreMesh(core_axis_name="c", subcore_axis_name="s")
    if nsrc == 3:
        scratch = [
            pltpu.VMEM((CHB,), jnp.int32),
            pltpu.VMEM((CHB,), jnp.int32),
            pltpu.VMEM((CHB,), jnp.int32),
            pltpu.VMEM((CHB, H), jnp.float32),
            pltpu.VMEM((CHB, H), jnp.float32),
            pltpu.VMEM((CHB, H), jnp.float32),
            pltpu.VMEM((RPW,), jnp.int32),
            pltpu.VMEM((NPB, 4, H), jnp.float32),
            pltpu.VMEM((NPB, 16), jnp.float32),
        ]
    else:
        scratch = [
            pltpu.VMEM((CHB,), jnp.int32),
            pltpu.VMEM((CHB,), jnp.int32),
            pltpu.VMEM((CHB, H), jnp.float32),
            pltpu.VMEM((RPW,), jnp.int32),
            pltpu.VMEM((NPB, 4, H), jnp.float32),
            pltpu.VMEM((NPB, 16), jnp.float32),
        ]
    return pl.kernel(
        functools.partial(_sc_stats_body, nsrc),
        out_type=(jax.ShapeDtypeStruct((NPAD, 4, H), jnp.float32),
                  jax.ShapeDtypeStruct((NPAD, 16), jnp.float32)),
        mesh=mesh,
        scratch_types=scratch,
    )


def _sc_pairsum_body(a_hbm, b_hbm, src_hbm, dst_hbm, out_hbm,
                     ia_v, ib_v, bufa_v, bufb_v):
    w = lax.axis_index("s") * 2 + lax.axis_index("c")
    base0 = w * EPW

    def chunk_body(k, _):
        base = base0 + k * CH2
        pltpu.sync_copy(src_hbm.at[pl.ds(base, CH2)], ia_v)
        pltpu.sync_copy(dst_hbm.at[pl.ds(base, CH2)], ib_v)
        pltpu.sync_copy(a_hbm.at[ia_v], bufa_v)
        pltpu.sync_copy(b_hbm.at[ib_v], bufb_v)

        def row_body(j, _):
            for t in range(8):
                s = pl.ds(16 * t, 16)
                bufa_v[j, s] = bufa_v[j, s] + bufb_v[j, s]
            return 0

        lax.fori_loop(0, CH2, row_body, 0)
        pltpu.sync_copy(bufa_v, out_hbm.at[pl.ds(base, CH2)])
        return 0

    lax.fori_loop(0, EPW // CH2, chunk_body, 0)


_sc_pairsum = pl.kernel(
    _sc_pairsum_body,
    out_type=jax.ShapeDtypeStruct((E, H), jnp.float32),
    mesh=plsc.VectorSubcoreMesh(core_axis_name="c", subcore_axis_name="s"),
    scratch_types=[
        pltpu.VMEM((CH2,), jnp.int32),
        pltpu.VMEM((CH2,), jnp.int32),
        pltpu.VMEM((CH2, H), jnp.float32),
        pltpu.VMEM((CH2, H), jnp.float32),
    ],
)


def _mm(a, w):
    return jnp.dot(a, w, preferred_element_type=jnp.float32)


def _tc_rows_call(body, n_rows, tile, out_dim, *arrays):
    grid = n_rows // tile
    in_specs = []
    for arr in arrays:
        if arr.shape[0] == n_rows:
            in_specs.append(
                pl.BlockSpec((tile,) + arr.shape[1:],
                             lambda i, _nd=arr.ndim: (i,) + (0,) * (_nd - 1)))
        else:
            in_specs.append(
                pl.BlockSpec(arr.shape, lambda i, _nd=arr.ndim: (0,) * _nd))
    return pl.pallas_call(
        body,
        grid=(grid,),
        in_specs=in_specs,
        out_specs=pl.BlockSpec((tile, out_dim), lambda i: (i, 0)),
        out_shape=jax.ShapeDtypeStruct((n_rows, out_dim), jnp.float32),
    )(*arrays)


def _node_in_body(x_ref, w_ref, b_ref, o_ref):
    o_ref[...] = jax.nn.relu(_mm(x_ref[...], w_ref[...]) + b_ref[...])


def _edge_c_body(ea_ref, we_ref, be_ref, w3_ref, b3_ref, o_ref):
    eh = jax.nn.relu(_mm(ea_ref[...], we_ref[...]) + be_ref[...])
    for l in range(3):
        o_ref[l] = _mm(eh, w3_ref[l]) + b3_ref[l]


def _ab_body(h_ref, w_ref, b_ref, o_ref):
    o_ref[...] = _mm(h_ref[...], w_ref[...]) + b_ref[...]


def _ec_body(g_ref, w2_ref, b2_ref, wse_ref, bse_ref, o_ref):
    m = _mm(jax.nn.relu(g_ref[...]), w2_ref[...]) + b2_ref[...]
    gate = jax.nn.sigmoid(_mm(m, wse_ref[...]) + bse_ref[...])
    o_ref[...] = m * gate


def _post_body(h_ref, sb_ref, db_ref, sc_ref, dc_ref, wp_ref, bp_ref, o_ref):
    h = h_ref[...]
    acc = _mm(h, wp_ref[0]) + bp_ref[...]
    for et, st_ref, dg_ref in ((0, sb_ref, db_ref), (1, sc_ref, dc_ref)):
        s = st_ref[:, 0, :]
        q = st_ref[:, 1, :]
        mx = st_ref[:, 2, :]
        mn = st_ref[:, 3, :]
        deg = dg_ref[:, 0][:, None]
        mask = deg > 0.0
        degc = jnp.maximum(deg, 1.0)
        mean = s / degc
        std = jnp.sqrt(jax.nn.relu(q / degc - mean * mean) + 1e-5)
        mx = jnp.where(mask, mx, 0.0)
        mn = jnp.where(mask, mn, 0.0)
        mean = jnp.where(mask, mean, 0.0)
        std = jnp.where(mask, std, 0.0)
        logd = jnp.log(deg + 1.0)
        safe_logd = jnp.where(mask, logd, 1.0)
        amp = logd / AVG_D_LOG
        att = jnp.where(mask, AVG_D_LOG / safe_logd, 0.0)
        base = 1 + 12 * et
        for si, stat in enumerate((mean, mx, mn, std)):
            acc = acc + _mm(stat, wp_ref[base + si])
            acc = acc + _mm(stat * amp, wp_ref[base + 4 + si])
            acc = acc + _mm(stat * att, wp_ref[base + 8 + si])
    o_ref[...] = acc + h


def _epi_body(h_ref, w1_ref, b1_ref, w2_ref, b2_ref,
              wr1_ref, br1_ref, wr2_ref, br2_ref, o_ref, acc_ref):
    i = pl.program_id(0)
    hf = _mm(jax.nn.relu(_mm(h_ref[...], w1_ref[...]) + b1_ref[...]),
             w2_ref[...]) + b2_ref[...]
    psum = jnp.sum(hf, axis=0, keepdims=True)
    pmax = jnp.max(hf, axis=0, keepdims=True)

    @pl.when(i == 0)
    def _():
        acc_ref[0:1, :] = psum
        acc_ref[1:2, :] = pmax

    @pl.when(i > 0)
    def _():
        acc_ref[0:1, :] = acc_ref[0:1, :] + psum
        acc_ref[1:2, :] = jnp.maximum(acc_ref[1:2, :], pmax)

    @pl.when(i == pl.num_programs(0) - 1)
    def _():
        ssum = acc_ref[0:1, :]
        smax = acc_ref[1:2, :]
        g = jnp.concatenate([ssum, ssum / N, smax], axis=1)
        r = jax.nn.relu(_mm(g, wr1_ref[...]) + br1_ref[...])
        o_ref[...] = _mm(r, wr2_ref[...]) + br2_ref[...]


def _prep_edges(ei):
    src, dst = ei[0], ei[1]
    perm = jnp.argsort(dst).astype(jnp.int32)
    src_s = src[perm].astype(jnp.int32)
    dst_s = dst[perm].astype(jnp.int32)
    rowptr = jnp.searchsorted(dst_s, jnp.arange(PADN, dtype=jnp.int32)
                              ).astype(jnp.int32)
    pad = jnp.zeros((PADE - E,), jnp.int32)
    return (jnp.concatenate([src_s, pad]), jnp.concatenate([dst_s, pad]),
            jnp.concatenate([perm, pad]), rowptr)


_sc_stats3 = _make_sc_stats(3)
_sc_stats1 = _make_sc_stats(1)


@jax.jit
def kernel(x, edge_attr, params, edge_index_bond, edge_index_complete):
    srcb_s, dstb_s, permb, rpb = _prep_edges(edge_index_bond)
    _, dstc_s, permc, rpc = _prep_edges(edge_index_complete)
    srcc = edge_index_complete[0].astype(jnp.int32)
    dstc = edge_index_complete[1].astype(jnp.int32)

    p = params
    Wn, bn = p['node_in'][0]
    We, be = p['edge_in'][0]

    h = _tc_rows_call(_node_in_body, N, 1000, H, x, Wn, bn[None, :])

    W3 = jnp.stack([p['layers'][l]['pretrans'][0][0][2 * H:3 * H]
                    for l in range(3)])
    b3 = jnp.stack([p['layers'][l]['pretrans'][0][1] for l in range(3)])
    Call = pl.pallas_call(
        _edge_c_body,
        grid=(E // 2000,),
        in_specs=[
            pl.BlockSpec((2000, 16), lambda i: (i, 0)),
            pl.BlockSpec((16, H), lambda i: (0, 0)),
            pl.BlockSpec((1, H), lambda i: (0, 0)),
            pl.BlockSpec((3, H, H), lambda i: (0, 0, 0)),
            pl.BlockSpec((3, 1, H), lambda i: (0, 0, 0)),
        ],
        out_specs=pl.BlockSpec((3, 2000, H), lambda i: (0, i, 0)),
        out_shape=jax.ShapeDtypeStruct((3, E, H), jnp.float32),
    )(edge_attr, We, be[None, :], W3, b3[:, None, :])

    Wcat_s, bcat_s, W2c_s, b2c_s, Wse_s, bse_s, Wp3_s, bp_s = [], [], [], \
        [], [], [], [], []
    for l in range(3):
        lp = p['layers'][l]
        Wpre = lp['pretrans'][0][0]
        Vpre, c1 = lp['pretrans_complete'][0]
        Wcat_s.append(jnp.concatenate(
            [Wpre[:H], Wpre[H:2 * H], Vpre[:H], Vpre[H:]], axis=1))
        bcat_s.append(jnp.concatenate(
            [jnp.zeros((2 * H,), jnp.float32), c1,
             jnp.zeros((H,), jnp.float32)]))
        W2c_s.append(lp['pretrans_complete'][1][0])
        b2c_s.append(lp['pretrans_complete'][1][1])
        Wse_s.append(lp['soft_edge'][0])
        bse_s.append(lp['soft_edge'][1])
        Wp3_s.append(lp['posttrans'][0][0].reshape(25, H, H))
        bp_s.append(lp['posttrans'][0][1])
    xs = tuple(jnp.stack(v) for v in
               (Wcat_s, bcat_s, W2c_s, b2c_s, Wse_s, bse_s, Wp3_s, bp_s)) \
        + (Call,)

    def layer_step(h, xs_l):
        Wcat, bcat, W2c, b2c, Wse, bse, Wp3, bp, C_l = xs_l
        ab = _tc_rows_call(_ab_body, N, 1000, 4 * H, h, Wcat, bcat[None, :])
        A, B = ab[:, :H], ab[:, H:2 * H]
        A2, B2 = ab[:, 2 * H:3 * H], ab[:, 3 * H:]

        stats_b, deg_b = _sc_stats3(A, B, C_l, srcb_s, dstb_s, permb, rpb)

        G = _sc_pairsum(A2, B2, srcc, dstc)
        EC = _tc_rows_call(_ec_body, E, 2000, H, G, W2c, b2c[None, :],
                           Wse, bse[None, :])
        stats_c, deg_c = _sc_stats1(EC, dstc_s, permc, rpc)

        hn = pl.pallas_call(
            _post_body,
            grid=(N // 1000,),
            in_specs=[
                pl.BlockSpec((1000, H), lambda i: (i, 0)),
                pl.BlockSpec((1000, 4, H), lambda i: (i, 0, 0)),
                pl.BlockSpec((1000, 16), lambda i: (i, 0)),
                pl.BlockSpec((1000, 4, H), lambda i: (i, 0, 0)),
                pl.BlockSpec((1000, 16), lambda i: (i, 0)),
                pl.BlockSpec((25, H, H), lambda i: (0, 0, 0)),
                pl.BlockSpec((1, H), lambda i: (0, 0)),
            ],
            out_specs=pl.BlockSpec((1000, H), lambda i: (i, 0)),
            out_shape=jax.ShapeDtypeStruct((N, H), jnp.float32),
        )(h, stats_b[:N], deg_b[:N], stats_c[:N], deg_c[:N],
          Wp3, bp[None, :])
        return hn, 0

    h, _ = lax.scan(layer_step, h, xs)

    Wo1, bo1 = p['node_out'][0]
    Wo2, bo2 = p['node_out'][1]
    Wr1, br1 = p['readout'][0]
    Wr2, br2 = p['readout'][1]
    out = pl.pallas_call(
        _epi_body,
        grid=(N // 1000,),
        in_specs=[
            pl.BlockSpec((1000, H), lambda i: (i, 0)),
            pl.BlockSpec((H, H), lambda i: (0, 0)),
            pl.BlockSpec((1, H), lambda i: (0, 0)),
            pl.BlockSpec((H, H), lambda i: (0, 0)),
            pl.BlockSpec((1, H), lambda i: (0, 0)),
            pl.BlockSpec((3 * H, H), lambda i: (0, 0)),
            pl.BlockSpec((1, H), lambda i: (0, 0)),
            pl.BlockSpec((H, 128), lambda i: (0, 0)),
            pl.BlockSpec((1, 128), lambda i: (0, 0)),
        ],
        out_specs=pl.BlockSpec((1, 128), lambda i: (0, 0)),
        out_shape=jax.ShapeDtypeStruct((1, 128), jnp.float32),
        scratch_shapes=[pltpu.VMEM((2, H), jnp.float32)],
    )(h, Wo1, bo1[None, :], Wo2, bo2[None, :],
      Wr1, br1[None, :], Wr2, br2[None, :])
    return out[0]


# speedup vs baseline: 2.5403x; 1.1261x over previous
import functools
import jax
import jax.numpy as jnp
from jax import lax
from jax.experimental import pallas as pl
from jax.experimental.pallas import tpu as pltpu
from jax.experimental.pallas import tpu_sc as plsc

N = 10000
E = 320000
H = 128
AVG_D_LOG = 1.0

NSC = 32
NPART = 96
NPB = 112
NPAD = NPART * NPB
CH = 96
CHB = CH + 16
CH2 = 400
EPW = E // NSC
RPW = 192
PADN = N + 1024
PADE = E + 256

F32MAX = 3.4e38


def _align8(i):
    return (i // 8) * 8


def _extract_i32(vref, idx):
    return vref[pl.ds(idx, 16)][0]


def _sc_stats_body(nsrc, *refs):
    if nsrc == 3:
        (a_hbm, b_hbm, c_hbm, srcs_hbm, dsts_hbm, perm_hbm, rp_hbm,
         out_hbm, deg_hbm,
         ia_v, ib_v, ic_v, bufa_v, bufb_v, bufc_v, rp_v, acc_v, deg_v,
         sem) = refs
    else:
        (c_hbm, dsts_hbm, perm_hbm, rp_hbm, out_hbm, deg_hbm,
         ib_v, ic_v, bufc_v, rp_v, acc_v, deg_v, sem) = refs

    w = lax.axis_index("s") * 2 + lax.axis_index("c")

    def round_body(r, _0):
        part = w * 3 + r
        n0 = part * NPB
        n1 = jnp.minimum(jnp.maximum(N, n0), n0 + NPB)
        rp_base = _align8(n0)
        pltpu.sync_copy(rp_hbm.at[pl.ds(rp_base, RPW)], rp_v)

        def rp(i, _rp_base=rp_base):
            return _extract_i32(rp_v, i - _rp_base)

        e_lo = rp(n0)
        e_hi = rp(n1)
        e_al = _align8(e_lo)
        nchunks = (e_hi - e_al + CH - 1) // CH

        def init_body(i, _):
            for t in range(8):
                s = pl.ds(16 * t, 16)
                acc_v[i, 0, s] = jnp.zeros((16,), jnp.float32)
                acc_v[i, 1, s] = jnp.zeros((16,), jnp.float32)
                acc_v[i, 2, s] = jnp.full((16,), -F32MAX, jnp.float32)
                acc_v[i, 3, s] = jnp.full((16,), F32MAX, jnp.float32)
            deg_v[i, pl.ds(0, 16)] = jnp.zeros((16,), jnp.float32)
            return 0

        lax.fori_loop(0, NPB, init_body, 0)

        def chunk_body(k, _, _n0=n0, _e_lo=e_lo, _e_hi=e_hi, _e_al=e_al):
            base = _e_al + k * CH
            if nsrc == 3:
                cps = [pltpu.async_copy(srcs_hbm.at[pl.ds(base, CHB)], ia_v,
                                        sem),
                       pltpu.async_copy(dsts_hbm.at[pl.ds(base, CHB)], ib_v,
                                        sem),
                       pltpu.async_copy(perm_hbm.at[pl.ds(base, CHB)], ic_v,
                                        sem)]
                for cp in cps:
                    cp.wait()
                gs = [pltpu.async_copy(a_hbm.at[ia_v], bufa_v, sem),
                      pltpu.async_copy(b_hbm.at[ib_v], bufb_v, sem),
                      pltpu.async_copy(c_hbm.at[ic_v], bufc_v, sem)]
                for g in gs:
                    g.wait()
            else:
                cps = [pltpu.async_copy(dsts_hbm.at[pl.ds(base, CHB)], ib_v,
                                        sem),
                       pltpu.async_copy(perm_hbm.at[pl.ds(base, CHB)], ic_v,
                                        sem)]
                for cp in cps:
                    cp.wait()
                pltpu.sync_copy(c_hbm.at[ic_v], bufc_v)
            j0 = jnp.maximum(_e_lo - base, 0)
            j1 = jnp.minimum(_e_hi - base, CH)

            def edge_body(j, _2):
                off = _extract_i32(ib_v, j) - _n0
                ones = jnp.full((16,), 1.0, jnp.float32)
                deg_v[off, pl.ds(0, 16)] = deg_v[off, pl.ds(0, 16)] + ones
                for t in range(8):
                    s = pl.ds(16 * t, 16)
                    m = bufc_v[j, s]
                    if nsrc == 3:
                        m = m + bufa_v[j, s]
                        m = m + bufb_v[j, s]
                    acc_v[off, 0, s] = acc_v[off, 0, s] + m
                    acc_v[off, 1, s] = acc_v[off, 1, s] + m * m
                    acc_v[off, 2, s] = jnp.maximum(acc_v[off, 2, s], m)
                    acc_v[off, 3, s] = jnp.minimum(acc_v[off, 3, s], m)
                return 0

            lax.fori_loop(j0, j1, edge_body, 0)
            return 0

        lax.fori_loop(0, nchunks, chunk_body, 0)
        pltpu.sync_copy(acc_v, out_hbm.at[pl.ds(n0, NPB)])
        pltpu.sync_copy(deg_v, deg_hbm.at[pl.ds(n0, NPB)])
        return 0

    lax.fori_loop(0, 3, round_body, 0)


def _make_sc_stats(nsrc):
    mesh = plsc.VectorSubcoreMesh(core_axis_name="c", subcore_axis_name="s")
    if nsrc == 3:
        scratch = [
            pltpu.VMEM((CHB,), jnp.int32),
            pltpu.VMEM((CHB,), jnp.int32),
            pltpu.VMEM((CHB,), jnp.int32),
            pltpu.VMEM((CHB, H), jnp.float32),
            pltpu.VMEM((CHB, H), jnp.float32),
            pltpu.VMEM((CHB, H), jnp.float32),
            pltpu.VMEM((RPW,), jnp.int32),
            pltpu.VMEM((NPB, 4, H), jnp.float32),
            pltpu.VMEM((NPB, 16), jnp.float32),
            pltpu.SemaphoreType.DMA,
        ]
    else:
        scratch = [
            pltpu.VMEM((CHB,), jnp.int32),
            pltpu.VMEM((CHB,), jnp.int32),
            pltpu.VMEM((CHB, H), jnp.float32),
            pltpu.VMEM((RPW,), jnp.int32),
            pltpu.VMEM((NPB, 4, H), jnp.float32),
            pltpu.VMEM((NPB, 16), jnp.float32),
            pltpu.SemaphoreType.DMA,
        ]
    return pl.kernel(
        functools.partial(_sc_stats_body, nsrc),
        out_type=(jax.ShapeDtypeStruct((NPAD, 4, H), jnp.float32),
                  jax.ShapeDtypeStruct((NPAD, 16), jnp.float32)),
        mesh=mesh,
        scratch_types=scratch,
    )


def _sc_pairsum_body(a_hbm, b_hbm, src_hbm, dst_hbm, out_hbm,
                     ia_v, ib_v, bufa_v, bufb_v, sem):
    w = lax.axis_index("s") * 2 + lax.axis_index("c")
    base0 = w * EPW

    def chunk_body(k, _):
        base = base0 + k * CH2
        cps = [pltpu.async_copy(src_hbm.at[pl.ds(base, CH2)], ia_v, sem),
               pltpu.async_copy(dst_hbm.at[pl.ds(base, CH2)], ib_v, sem)]
        for cp in cps:
            cp.wait()
        gs = [pltpu.async_copy(a_hbm.at[ia_v], bufa_v, sem),
              pltpu.async_copy(b_hbm.at[ib_v], bufb_v, sem)]
        for g in gs:
            g.wait()

        def row_body(j, _):
            for t in range(8):
                s = pl.ds(16 * t, 16)
                bufa_v[j, s] = bufa_v[j, s] + bufb_v[j, s]
            return 0

        lax.fori_loop(0, CH2, row_body, 0)
        pltpu.sync_copy(bufa_v, out_hbm.at[pl.ds(base, CH2)])
        return 0

    lax.fori_loop(0, EPW // CH2, chunk_body, 0)


_sc_pairsum = pl.kernel(
    _sc_pairsum_body,
    out_type=jax.ShapeDtypeStruct((E, H), jnp.float32),
    mesh=plsc.VectorSubcoreMesh(core_axis_name="c", subcore_axis_name="s"),
    scratch_types=[
        pltpu.VMEM((CH2,), jnp.int32),
        pltpu.VMEM((CH2,), jnp.int32),
        pltpu.VMEM((CH2, H), jnp.float32),
        pltpu.VMEM((CH2, H), jnp.float32),
        pltpu.SemaphoreType.DMA,
    ],
)


def _mm(a, w):
    return jnp.dot(a, w, preferred_element_type=jnp.float32)


def _tc_rows_call(body, n_rows, tile, out_dim, *arrays):
    grid = n_rows // tile
    in_specs = []
    for arr in arrays:
        if arr.shape[0] == n_rows:
            in_specs.append(
                pl.BlockSpec((tile,) + arr.shape[1:],
                             lambda i, _nd=arr.ndim: (i,) + (0,) * (_nd - 1)))
        else:
            in_specs.append(
                pl.BlockSpec(arr.shape, lambda i, _nd=arr.ndim: (0,) * _nd))
    return pl.pallas_call(
        body,
        grid=(grid,),
        in_specs=in_specs,
        out_specs=pl.BlockSpec((tile, out_dim), lambda i: (i, 0)),
        out_shape=jax.ShapeDtypeStruct((n_rows, out_dim), jnp.float32),
    )(*arrays)


def _node_in_body(x_ref, w_ref, b_ref, o_ref):
    o_ref[...] = jax.nn.relu(_mm(x_ref[...], w_ref[...]) + b_ref[...])


def _edge_c_body(ea_ref, we_ref, be_ref, w3_ref, b3_ref, o_ref):
    eh = jax.nn.relu(_mm(ea_ref[...], we_ref[...]) + be_ref[...])
    for l in range(3):
        o_ref[l] = _mm(eh, w3_ref[l]) + b3_ref[l]


def _ab_body(h_ref, w_ref, b_ref, o_ref):
    o_ref[...] = _mm(h_ref[...], w_ref[...]) + b_ref[...]


def _ec_body(g_ref, w2_ref, b2_ref, wse_ref, bse_ref, o_ref):
    m = _mm(jax.nn.relu(g_ref[...]), w2_ref[...]) + b2_ref[...]
    gate = jax.nn.sigmoid(_mm(m, wse_ref[...]) + bse_ref[...])
    o_ref[...] = m * gate


def _post_body(h_ref, sb_ref, db_ref, sc_ref, dc_ref, wp_ref, bp_ref, o_ref):
    h = h_ref[...]
    acc = _mm(h, wp_ref[0]) + bp_ref[...]
    for et, st_ref, dg_ref in ((0, sb_ref, db_ref), (1, sc_ref, dc_ref)):
        s = st_ref[:, 0, :]
        q = st_ref[:, 1, :]
        mx = st_ref[:, 2, :]
        mn = st_ref[:, 3, :]
        deg = dg_ref[:, 0][:, None]
        mask = deg > 0.0
        degc = jnp.maximum(deg, 1.0)
        mean = s / degc
        std = jnp.sqrt(jax.nn.relu(q / degc - mean * mean) + 1e-5)
        mx = jnp.where(mask, mx, 0.0)
        mn = jnp.where(mask, mn, 0.0)
        mean = jnp.where(mask, mean, 0.0)
        std = jnp.where(mask, std, 0.0)
        logd = jnp.log(deg + 1.0)
        safe_logd = jnp.where(mask, logd, 1.0)
        amp = logd / AVG_D_LOG
        att = jnp.where(mask, AVG_D_LOG / safe_logd, 0.0)
        base = 1 + 12 * et
        for si, stat in enumerate((mean, mx, mn, std)):
            acc = acc + _mm(stat, wp_ref[base + si])
            acc = acc + _mm(stat * amp, wp_ref[base + 4 + si])
            acc = acc + _mm(stat * att, wp_ref[base + 8 + si])
    o_ref[...] = acc + h


def _epi_body(h_ref, w1_ref, b1_ref, w2_ref, b2_ref,
              wr1_ref, br1_ref, wr2_ref, br2_ref, o_ref, acc_ref):
    i = pl.program_id(0)
    hf = _mm(jax.nn.relu(_mm(h_ref[...], w1_ref[...]) + b1_ref[...]),
             w2_ref[...]) + b2_ref[...]
    psum = jnp.sum(hf, axis=0, keepdims=True)
    pmax = jnp.max(hf, axis=0, keepdims=True)

    @pl.when(i == 0)
    def _():
        acc_ref[0:1, :] = psum
        acc_ref[1:2, :] = pmax

    @pl.when(i > 0)
    def _():
        acc_ref[0:1, :] = acc_ref[0:1, :] + psum
        acc_ref[1:2, :] = jnp.maximum(acc_ref[1:2, :], pmax)

    @pl.when(i == pl.num_programs(0) - 1)
    def _():
        ssum = acc_ref[0:1, :]
        smax = acc_ref[1:2, :]
        g = jnp.concatenate([ssum, ssum / N, smax], axis=1)
        r = jax.nn.relu(_mm(g, wr1_ref[...]) + br1_ref[...])
        o_ref[...] = _mm(r, wr2_ref[...]) + br2_ref[...]


def _prep_edges(ei):
    src, dst = ei[0], ei[1]
    perm = jnp.argsort(dst).astype(jnp.int32)
    src_s = src[perm].astype(jnp.int32)
    dst_s = dst[perm].astype(jnp.int32)
    rowptr = jnp.searchsorted(dst_s, jnp.arange(PADN, dtype=jnp.int32)
                              ).astype(jnp.int32)
    pad = jnp.zeros((PADE - E,), jnp.int32)
    return (jnp.concatenate([src_s, pad]), jnp.concatenate([dst_s, pad]),
            jnp.concatenate([perm, pad]), rowptr)


_sc_stats3 = _make_sc_stats(3)
_sc_stats1 = _make_sc_stats(1)


@jax.jit
def kernel(x, edge_attr, params, edge_index_bond, edge_index_complete):
    srcb_s, dstb_s, permb, rpb = _prep_edges(edge_index_bond)
    _, dstc_s, permc, rpc = _prep_edges(edge_index_complete)
    srcc = edge_index_complete[0].astype(jnp.int32)
    dstc = edge_index_complete[1].astype(jnp.int32)

    p = params
    Wn, bn = p['node_in'][0]
    We, be = p['edge_in'][0]

    h = _tc_rows_call(_node_in_body, N, 1000, H, x, Wn, bn[None, :])

    W3 = jnp.stack([p['layers'][l]['pretrans'][0][0][2 * H:3 * H]
                    for l in range(3)])
    b3 = jnp.stack([p['layers'][l]['pretrans'][0][1] for l in range(3)])
    Call = pl.pallas_call(
        _edge_c_body,
        grid=(E // 2000,),
        in_specs=[
            pl.BlockSpec((2000, 16), lambda i: (i, 0)),
            pl.BlockSpec((16, H), lambda i: (0, 0)),
            pl.BlockSpec((1, H), lambda i: (0, 0)),
            pl.BlockSpec((3, H, H), lambda i: (0, 0, 0)),
            pl.BlockSpec((3, 1, H), lambda i: (0, 0, 0)),
        ],
        out_specs=pl.BlockSpec((3, 2000, H), lambda i: (0, i, 0)),
        out_shape=jax.ShapeDtypeStruct((3, E, H), jnp.float32),
    )(edge_attr, We, be[None, :], W3, b3[:, None, :])

    Wcat_s, bcat_s, W2c_s, b2c_s, Wse_s, bse_s, Wp3_s, bp_s = [], [], [], \
        [], [], [], [], []
    for l in range(3):
        lp = p['layers'][l]
        Wpre = lp['pretrans'][0][0]
        Vpre, c1 = lp['pretrans_complete'][0]
        Wcat_s.append(jnp.concatenate(
            [Wpre[:H], Wpre[H:2 * H], Vpre[:H], Vpre[H:]], axis=1))
        bcat_s.append(jnp.concatenate(
            [jnp.zeros((2 * H,), jnp.float32), c1,
             jnp.zeros((H,), jnp.float32)]))
        W2c_s.append(lp['pretrans_complete'][1][0])
        b2c_s.append(lp['pretrans_complete'][1][1])
        Wse_s.append(lp['soft_edge'][0])
        bse_s.append(lp['soft_edge'][1])
        Wp3_s.append(lp['posttrans'][0][0].reshape(25, H, H))
        bp_s.append(lp['posttrans'][0][1])
    xs = tuple(jnp.stack(v) for v in
               (Wcat_s, bcat_s, W2c_s, b2c_s, Wse_s, bse_s, Wp3_s, bp_s)) \
        + (Call,)

    def layer_step(h, xs_l):
        Wcat, bcat, W2c, b2c, Wse, bse, Wp3, bp, C_l = xs_l
        ab = _tc_rows_call(_ab_body, N, 1000, 4 * H, h, Wcat, bcat[None, :])
        A, B = ab[:, :H], ab[:, H:2 * H]
        A2, B2 = ab[:, 2 * H:3 * H], ab[:, 3 * H:]

        stats_b, deg_b = _sc_stats3(A, B, C_l, srcb_s, dstb_s, permb, rpb)

        G = _sc_pairsum(A2, B2, srcc, dstc)
        EC = _tc_rows_call(_ec_body, E, 2000, H, G, W2c, b2c[None, :],
                           Wse, bse[None, :])
        stats_c, deg_c = _sc_stats1(EC, dstc_s, permc, rpc)

        hn = pl.pallas_call(
            _post_body,
            grid=(N // 1000,),
            in_specs=[
                pl.BlockSpec((1000, H), lambda i: (i, 0)),
                pl.BlockSpec((1000, 4, H), lambda i: (i, 0, 0)),
                pl.BlockSpec((1000, 16), lambda i: (i, 0)),
                pl.BlockSpec((1000, 4, H), lambda i: (i, 0, 0)),
                pl.BlockSpec((1000, 16), lambda i: (i, 0)),
                pl.BlockSpec((25, H, H), lambda i: (0, 0, 0)),
                pl.BlockSpec((1, H), lambda i: (0, 0)),
            ],
            out_specs=pl.BlockSpec((1000, H), lambda i: (i, 0)),
            out_shape=jax.ShapeDtypeStruct((N, H), jnp.float32),
        )(h, stats_b[:N], deg_b[:N], stats_c[:N], deg_c[:N],
          Wp3, bp[None, :])
        return hn, 0

    h, _ = lax.scan(layer_step, h, xs)

    Wo1, bo1 = p['node_out'][0]
    Wo2, bo2 = p['node_out'][1]
    Wr1, br1 = p['readout'][0]
    Wr2, br2 = p['readout'][1]
    out = pl.pallas_call(
        _epi_body,
        grid=(N // 1000,),
        in_specs=[
            pl.BlockSpec((1000, H), lambda i: (i, 0)),
            pl.BlockSpec((H, H), lambda i: (0, 0)),
            pl.BlockSpec((1, H), lambda i: (0, 0)),
            pl.BlockSpec((H, H), lambda i: (0, 0)),
            pl.BlockSpec((1, H), lambda i: (0, 0)),
            pl.BlockSpec((3 * H, H), lambda i: (0, 0)),
            pl.BlockSpec((1, H), lambda i: (0, 0)),
            pl.BlockSpec((H, 128), lambda i: (0, 0)),
            pl.BlockSpec((1, 128), lambda i: (0, 0)),
        ],
        out_specs=pl.BlockSpec((1, 128), lambda i: (0, 0)),
        out_shape=jax.ShapeDtypeStruct((1, 128), jnp.float32),
        scratch_shapes=[pltpu.VMEM((2, H), jnp.float32)],
    )(h, Wo1, bo1[None, :], Wo2, bo2[None, :],
      Wr1, br1[None, :], Wr2, br2[None, :])
    return out[0]
